# Initial kernel scaffold; baseline (speedup 1.0000x reference)
import functools

import jax
import jax.numpy as jnp
from jax.experimental import pallas as pl
from jax.experimental.pallas import tpu as pltpu


def _routing(note_type_pos, n_experts, blk, num_blocks):
    t = note_type_pos.shape[0]
    e = note_type_pos.astype(jnp.int32)
    order = jnp.argsort(e, stable=True)
    es = e[order]
    counts = jnp.bincount(e, length=n_experts).astype(jnp.int32)
    blocks_per = (counts + blk - 1) // blk
    starts_blk = jnp.concatenate(
        [jnp.zeros((1,), jnp.int32), jnp.cumsum(blocks_per)[:-1].astype(jnp.int32)])
    starts_row = starts_blk * blk
    cum_counts = jnp.concatenate(
        [jnp.zeros((1,), jnp.int32), jnp.cumsum(counts)[:-1].astype(jnp.int32)])
    rank = jnp.arange(t, dtype=jnp.int32) - cum_counts[es]
    slot_sorted = starts_row[es] + rank
    slot_tok = jnp.zeros((t,), jnp.int32).at[order].set(slot_sorted)
    p = num_blocks * blk
    src = jnp.zeros((p,), jnp.int32).at[slot_sorted].set(order)
    block_expert = jnp.clip(
        jnp.searchsorted(starts_blk, jnp.arange(num_blocks, dtype=jnp.int32),
                         side="right").astype(jnp.int32) - 1,
        0, n_experts - 1)
    return slot_tok, src, block_expert


def _gmm1_body(be_ref, x_ref, w1_ref, b1_ref, h_ref):
    acc = jnp.dot(x_ref[...], w1_ref[0], preferred_element_type=jnp.float32)
    h_ref[...] = jnp.maximum(acc + b1_ref[0], 0.0)


def _gmm2_body(be_ref, h_ref, w2_ref, b2_ref, o_ref):
    acc = jnp.dot(h_ref[...], w2_ref[0], preferred_element_type=jnp.float32)
    o_ref[...] = acc + b2_ref[0]


def kernel(x, note_type_pos, W1, b1, W2, b2):
    t, h_dim = x.shape
    n, _, ff = W1.shape
    blk = 256 if t >= 256 else 8
    fft = 2048 if ff >= 2048 else ff
    ht = 512 if h_dim >= 512 else h_dim
    num_blocks = (t + n * blk) // blk
    p = num_blocks * blk

    slot_tok, src, block_expert = _routing(note_type_pos, n, blk, num_blocks)

    xs = x[src]

    nj1 = ff // fft
    hs = pl.pallas_call(
        _gmm1_body,
        grid_spec=pltpu.PrefetchScalarGridSpec(
            num_scalar_prefetch=1,
            grid=(nj1, num_blocks),
            in_specs=[
                pl.BlockSpec((blk, h_dim), lambda jf, i, be: (i, 0)),
                pl.BlockSpec((1, h_dim, fft), lambda jf, i, be: (be[i], 0, jf)),
                pl.BlockSpec((1, fft), lambda jf, i, be: (be[i], jf)),
            ],
            out_specs=pl.BlockSpec((blk, fft), lambda jf, i, be: (i, jf)),
        ),
        out_shape=jax.ShapeDtypeStruct((p, ff), jnp.float32),
    )(block_expert, xs, W1, b1)

    nj2 = h_dim // ht
    outs = pl.pallas_call(
        _gmm2_body,
        grid_spec=pltpu.PrefetchScalarGridSpec(
            num_scalar_prefetch=1,
            grid=(nj2, num_blocks),
            in_specs=[
                pl.BlockSpec((blk, ff), lambda jh, i, be: (i, 0)),
                pl.BlockSpec((1, ff, ht), lambda jh, i, be: (be[i], 0, jh)),
                pl.BlockSpec((1, ht), lambda jh, i, be: (be[i], jh)),
            ],
            out_specs=pl.BlockSpec((blk, ht), lambda jh, i, be: (i, jh)),
        ),
        out_shape=jax.ShapeDtypeStruct((p, h_dim), jnp.float32),
    )(block_expert, hs, W2, b2)

    return outs[slot_tok]


# speedup vs baseline: 3.2836x; 3.2836x over previous
import functools

import jax
import jax.numpy as jnp
from jax.experimental import pallas as pl
from jax.experimental.pallas import tpu as pltpu


def _routing(note_type_pos, n_experts, blk, num_blocks):
    t = note_type_pos.shape[0]
    e = note_type_pos.astype(jnp.int32)
    order = jnp.argsort(e, stable=True)
    es = e[order]
    counts = jnp.bincount(e, length=n_experts).astype(jnp.int32)
    blocks_per = (counts + blk - 1) // blk
    starts_blk = jnp.concatenate(
        [jnp.zeros((1,), jnp.int32), jnp.cumsum(blocks_per)[:-1].astype(jnp.int32)])
    starts_row = starts_blk * blk
    cum_counts = jnp.concatenate(
        [jnp.zeros((1,), jnp.int32), jnp.cumsum(counts)[:-1].astype(jnp.int32)])
    rank = jnp.arange(t, dtype=jnp.int32) - cum_counts[es]
    slot_sorted = starts_row[es] + rank
    slot_tok = jnp.zeros((t,), jnp.int32).at[order].set(slot_sorted)
    p = num_blocks * blk
    src = jnp.zeros((p,), jnp.int32).at[slot_sorted].set(order)
    block_expert = jnp.clip(
        jnp.searchsorted(starts_blk, jnp.arange(num_blocks, dtype=jnp.int32),
                         side="right").astype(jnp.int32) - 1,
        0, n_experts - 1)
    return slot_tok, src, block_expert


def _gmm1_body(be_ref, x_ref, w1_ref, b1_ref, h_ref):
    acc = jnp.dot(x_ref[...], w1_ref[0], preferred_element_type=jnp.float32)
    h_ref[...] = jnp.maximum(acc + b1_ref[0], 0.0)


def _gmm2_body(be_ref, h_ref, w2_ref, b2_ref, o_ref):
    acc = jnp.dot(h_ref[...], w2_ref[0], preferred_element_type=jnp.float32)
    o_ref[...] = acc + b2_ref[0]


def _b3d(b):
    return b[:, None, :]


def kernel(x, note_type_pos, W1, b1, W2, b2):
    t, h_dim = x.shape
    n, _, ff = W1.shape
    blk = 256 if t >= 256 else 8
    fft = 2048 if ff >= 2048 else ff
    ht = 512 if h_dim >= 512 else h_dim
    num_blocks = (t + n * blk) // blk
    p = num_blocks * blk

    slot_tok, src, block_expert = _routing(note_type_pos, n, blk, num_blocks)

    xs = x[src]

    nj1 = ff // fft
    hs = pl.pallas_call(
        _gmm1_body,
        grid_spec=pltpu.PrefetchScalarGridSpec(
            num_scalar_prefetch=1,
            grid=(nj1, num_blocks),
            in_specs=[
                pl.BlockSpec((blk, h_dim), lambda jf, i, be: (i, 0)),
                pl.BlockSpec((1, h_dim, fft), lambda jf, i, be: (be[i], 0, jf)),
                pl.BlockSpec((1, 1, fft), lambda jf, i, be: (be[i], 0, jf)),
            ],
            out_specs=pl.BlockSpec((blk, fft), lambda jf, i, be: (i, jf)),
        ),
        out_shape=jax.ShapeDtypeStruct((p, ff), jnp.float32),
    )(block_expert, xs, W1, _b3d(b1))

    nj2 = h_dim // ht
    outs = pl.pallas_call(
        _gmm2_body,
        grid_spec=pltpu.PrefetchScalarGridSpec(
            num_scalar_prefetch=1,
            grid=(nj2, num_blocks),
            in_specs=[
                pl.BlockSpec((blk, ff), lambda jh, i, be: (i, 0)),
                pl.BlockSpec((1, ff, ht), lambda jh, i, be: (be[i], 0, jh)),
                pl.BlockSpec((1, 1, ht), lambda jh, i, be: (be[i], 0, jh)),
            ],
            out_specs=pl.BlockSpec((blk, ht), lambda jh, i, be: (i, jh)),
        ),
        out_shape=jax.ShapeDtypeStruct((p, h_dim), jnp.float32),
    )(block_expert, hs, W2, _b3d(b2))

    return outs[slot_tok]
